# Initial kernel scaffold; baseline (speedup 1.0000x reference)
import jax
import jax.numpy as jnp
from jax.experimental import pallas as pl


def kernel(graph_nodes, graph_edge_links, mask, current_ad, W_enc, b_enc, W_self1, W_nbr1, b1, W_self2, W_nbr2, b2, W_ad, b_ad):
    raise NotImplementedError("write your pallas kernel here")


# speedup vs baseline: 6.6811x; 6.6811x over previous
import functools
import math

import jax
import jax.numpy as jnp
from jax import lax
from jax.experimental import pallas as pl
from jax.experimental.pallas import tpu as pltpu
import jax.experimental.pallas.tpu_sc as plsc

_NC = 2
_NS = 16
_DW = 128


def _pick_chunk(edges_per_worker):
    k = 0
    for cand in range(8, 129, 8):
        if edges_per_worker % cand == 0:
            k = cand
    return k


def _sc_mesh():
    return plsc.VectorSubcoreMesh(core_axis_name="c", subcore_axis_name="s",
                                  num_cores=_NC, num_subcores=_NS)


def _make_sc_aggregate(NP, H, CH, K):
    RPT = NP // _NS

    def body(h_hbm, src_hbm, dst_hbm, znh_hbm,
             agg_out, srcv, dstv, rows, aggsp, sem):
        c = lax.axis_index("c")
        s = lax.axis_index("s")
        wid = c * _NS + s

        pltpu.sync_copy(znh_hbm.at[pl.ds(s * RPT, RPT)],
                        aggsp.at[pl.ds(s * RPT, RPT)])
        pltpu.sync_copy(src_hbm.at[wid], srcv)
        pltpu.sync_copy(dst_hbm.at[wid], dstv)
        plsc.subcore_barrier()

        @pl.loop(0, CH)
        def _chunk(j):
            pltpu.async_copy(h_hbm.at[srcv.at[j]], rows, sem).wait()
            pltpu.sync_copy(rows, aggsp.at[dstv.at[j]], add=True)

        plsc.subcore_barrier()
        pltpu.sync_copy(aggsp.at[pl.ds(s * RPT, RPT)],
                        agg_out.at[c, pl.ds(s * RPT, RPT)])

    return pl.kernel(
        body,
        out_type=jax.ShapeDtypeStruct((_NC, NP, H), jnp.float32),
        mesh=_sc_mesh(),
        scratch_types=(
            pltpu.VMEM((CH, K), jnp.int32),
            pltpu.VMEM((CH, K), jnp.int32),
            pltpu.VMEM((K, H), jnp.float32),
            pltpu.VMEM_SHARED((NP, H), jnp.float32),
            pltpu.SemaphoreType.DMA,
        ))


def _make_sc_degree(NP, CH, K):
    RPT = NP // _NS

    def body(dst_hbm, znd_hbm, ones_hbm, deg_out, dstv, onesv, degsp):
        c = lax.axis_index("c")
        s = lax.axis_index("s")
        wid = c * _NS + s
        pltpu.sync_copy(znd_hbm.at[pl.ds(s * RPT, RPT)],
                        degsp.at[pl.ds(s * RPT, RPT)])
        pltpu.sync_copy(dst_hbm.at[wid], dstv)
        pltpu.sync_copy(ones_hbm, onesv)
        plsc.subcore_barrier()

        @pl.loop(0, CH)
        def _chunk(j):
            pltpu.sync_copy(onesv, degsp.at[dstv.at[j]], add=True)

        plsc.subcore_barrier()
        pltpu.sync_copy(degsp.at[pl.ds(s * RPT, RPT)],
                        deg_out.at[c, pl.ds(s * RPT, RPT)])

    return pl.kernel(
        body,
        out_type=jax.ShapeDtypeStruct((_NC, NP, _DW), jnp.float32),
        mesh=_sc_mesh(),
        scratch_types=(
            pltpu.VMEM((CH, K), jnp.int32),
            pltpu.VMEM((K, _DW), jnp.float32),
            pltpu.VMEM_SHARED((NP, _DW), jnp.float32),
        ))


def _enc_body(x_ref, w_ref, b_ref, o_ref):
    o_ref[...] = jnp.maximum(
        jnp.dot(x_ref[...], w_ref[...], preferred_element_type=jnp.float32)
        + b_ref[...], 0.0)


def _tc_encode(x, w, b, R):
    N, D = x.shape
    H = w.shape[1]
    return pl.pallas_call(
        _enc_body,
        grid=(N // R,),
        in_specs=[
            pl.BlockSpec((R, D), lambda i: (i, 0)),
            pl.BlockSpec((D, H), lambda i: (0, 0)),
            pl.BlockSpec((1, H), lambda i: (0, 0)),
        ],
        out_specs=pl.BlockSpec((R, H), lambda i: (i, 0)),
        out_shape=jax.ShapeDtypeStruct((N, H), jnp.float32),
    )(x, w, b.reshape(1, H))


def _gnn_update(h_ref, a0_ref, a1_ref, d0_ref, d1_ref, ws_ref, wn_ref, b_ref):
    deg = jnp.maximum(d0_ref[...][:, :1] + d1_ref[...][:, :1], 1.0)
    agg = (a0_ref[...] + a1_ref[...]) / deg
    return jnp.maximum(
        jnp.dot(h_ref[...], ws_ref[...], preferred_element_type=jnp.float32)
        + jnp.dot(agg, wn_ref[...], preferred_element_type=jnp.float32)
        + b_ref[...], 0.0)


def _layer_body(h_ref, a0_ref, a1_ref, d0_ref, d1_ref, ws_ref, wn_ref, b_ref,
                o_ref):
    o_ref[...] = _gnn_update(h_ref, a0_ref, a1_ref, d0_ref, d1_ref,
                             ws_ref, wn_ref, b_ref)


def _layer_specs(R, H):
    return [
        pl.BlockSpec((R, H), lambda i: (i, 0)),
        pl.BlockSpec((R, H), lambda i: (i, 0)),
        pl.BlockSpec((R, H), lambda i: (i, 0)),
        pl.BlockSpec((R, _DW), lambda i: (i, 0)),
        pl.BlockSpec((R, _DW), lambda i: (i, 0)),
        pl.BlockSpec((H, H), lambda i: (0, 0)),
        pl.BlockSpec((H, H), lambda i: (0, 0)),
        pl.BlockSpec((1, H), lambda i: (0, 0)),
    ]


def _tc_layer(h, aggp, degp, ws, wn, b, R):
    N, H = h.shape
    return pl.pallas_call(
        _layer_body,
        grid=(N // R,),
        in_specs=_layer_specs(R, H),
        out_specs=pl.BlockSpec((R, H), lambda i: (i, 0)),
        out_shape=jax.ShapeDtypeStruct((N, H), jnp.float32),
    )(h, aggp[0], aggp[1], degp[0], degp[1], ws, wn, b.reshape(1, H))


def _final_body(h_ref, a0_ref, a1_ref, d0_ref, d1_ref, ws_ref, wn_ref, b_ref,
                ad_ref, wad_ref, bad_ref, mk_ref, o_ref):
    h3 = _gnn_update(h_ref, a0_ref, a1_ref, d0_ref, d1_ref,
                     ws_ref, wn_ref, b_ref)
    H = h3.shape[1]
    ad = jnp.maximum(
        jnp.dot(ad_ref[...], wad_ref[...], preferred_element_type=jnp.float32)
        + bad_ref[...], 0.0)
    logits = lax.dot_general(ad, h3, (((1,), (1,)), ((), ())),
                             preferred_element_type=jnp.float32)
    logits = logits * (1.0 / math.sqrt(H))
    o_ref[...] = jnp.where(mk_ref[...][0] > 0.0, logits,
                           jnp.float32(-1e9))[None]


def _tc_final(h, aggp, degp, ws, wn, b, ad, wad, bad, mask_f, R):
    N, H = h.shape
    A = ad.shape[1]
    G = N // R
    specs = _layer_specs(R, H) + [
        pl.BlockSpec((1, A), lambda i: (0, 0)),
        pl.BlockSpec((A, H), lambda i: (0, 0)),
        pl.BlockSpec((1, H), lambda i: (0, 0)),
        pl.BlockSpec((1, 1, R), lambda i: (i, 0, 0)),
    ]
    out = pl.pallas_call(
        _final_body,
        grid=(G,),
        in_specs=specs,
        out_specs=pl.BlockSpec((1, 1, R), lambda i: (i, 0, 0)),
        out_shape=jax.ShapeDtypeStruct((G, 1, R), jnp.float32),
    )(h, aggp[0], aggp[1], degp[0], degp[1], ws, wn, b.reshape(1, H),
      ad, wad, bad.reshape(1, H), mask_f.reshape(G, 1, R))
    return out.reshape(N)


def kernel(graph_nodes, graph_edge_links, mask, current_ad, W_enc, b_enc,
           W_self1, W_nbr1, b1, W_self2, W_nbr2, b2, W_ad, b_ad):
    B, N, D = graph_nodes.shape
    H = W_enc.shape[1]
    E = graph_edge_links.shape[2]
    NW = _NC * _NS
    assert E % NW == 0 and N % _NS == 0
    EW = E // NW
    K = _pick_chunk(EW)
    CH = EW // K
    R = 2000 if N % 2000 == 0 else 8
    NP = ((N + 127) // 128) * 128

    znh = jnp.zeros((NP, H), jnp.float32)
    znd = jnp.zeros((NP, _DW), jnp.float32)
    ones = jnp.ones((K, _DW), jnp.float32)

    sc_agg = _make_sc_aggregate(NP, H, CH, K)
    sc_deg = _make_sc_degree(NP, CH, K)

    outs = []
    for bi in range(B):
        x = graph_nodes[bi]
        src2 = graph_edge_links[bi, 0].reshape(NW, CH, K)
        dst2 = graph_edge_links[bi, 1].reshape(NW, CH, K)
        mask_f = mask[bi].astype(jnp.float32)

        h1 = _tc_encode(x, W_enc, b_enc, R)
        degp = sc_deg(dst2, znd, ones)
        aggp1 = sc_agg(h1, src2, dst2, znh)
        h2 = _tc_layer(h1, aggp1, degp, W_self1, W_nbr1, b1, R)
        aggp2 = sc_agg(h2, src2, dst2, znh)
        logits = _tc_final(h2, aggp2, degp, W_self2, W_nbr2, b2,
                           current_ad[bi:bi + 1], W_ad, b_ad, mask_f, R)
        outs.append(logits)
    return jnp.stack(outs, axis=0)


# speedup vs baseline: 9.6309x; 1.4415x over previous
import functools
import math

import jax
import jax.numpy as jnp
from jax import lax
from jax.experimental import pallas as pl
from jax.experimental.pallas import tpu as pltpu
import jax.experimental.pallas.tpu_sc as plsc

_NC = 2
_NS = 16
_DW = 128


def _pick_chunk(edges_per_worker):
    k = 0
    for cand in range(8, 129, 8):
        if edges_per_worker % cand == 0:
            k = cand
    return k


def _sc_mesh():
    return plsc.VectorSubcoreMesh(core_axis_name="c", subcore_axis_name="s",
                                  num_cores=_NC, num_subcores=_NS)


def _make_sc_aggregate(NP, H, CH, K):
    RPT = NP // _NS

    def body(h_hbm, pk_hbm, znh_hbm,
             agg_out, pkv, srcc0, srcc1, dstc0, dstc1, rows0, rows1,
             aggsp, sem0, sem1):
        c = lax.axis_index("c")
        s = lax.axis_index("s")
        wid = c * _NS + s
        rows = (rows0, rows1)
        sems = (sem0, sem1)
        srcc = (srcc0, srcc1)
        dstc = (dstc0, dstc1)

        pltpu.sync_copy(znh_hbm.at[pl.ds(s * RPT, RPT)],
                        aggsp.at[pl.ds(s * RPT, RPT)])
        pltpu.sync_copy(pk_hbm.at[wid], pkv)
        plsc.subcore_barrier()

        def unpack(jj, b):
            for t in range(K // 16):
                v = pkv[jj, pl.ds(t * 16, 16)]
                srcc[b][pl.ds(t * 16, 16)] = lax.bitwise_and(v, 0xFFFF)
                dstc[b][pl.ds(t * 16, 16)] = lax.shift_right_logical(v, 16)

        for b in range(2):
            unpack(b, b)
            pltpu.async_copy(h_hbm.at[srcc[b]], rows[b], sems[b])

        @pl.loop(0, CH, step=2)
        def _chunk(j):
            for b in range(2):
                jj = j + b

                @pl.when(jj < CH)
                def _():
                    pltpu.make_async_copy(h_hbm.at[srcc[b]],
                                          rows[b], sems[b]).wait()
                    pltpu.sync_copy(rows[b], aggsp.at[dstc[b]], add=True)

                    @pl.when(jj + 2 < CH)
                    def _():
                        unpack(jj + 2, b)
                        pltpu.async_copy(h_hbm.at[srcc[b]], rows[b], sems[b])

        plsc.subcore_barrier()
        pltpu.sync_copy(aggsp.at[pl.ds(s * RPT, RPT)],
                        agg_out.at[c, pl.ds(s * RPT, RPT)])

    return pl.kernel(
        body,
        out_type=jax.ShapeDtypeStruct((_NC, NP, H), jnp.float32),
        mesh=_sc_mesh(),
        scratch_types=(
            pltpu.VMEM((CH, K), jnp.int32),
            pltpu.VMEM((K,), jnp.int32),
            pltpu.VMEM((K,), jnp.int32),
            pltpu.VMEM((K,), jnp.int32),
            pltpu.VMEM((K,), jnp.int32),
            pltpu.VMEM((K, H), jnp.float32),
            pltpu.VMEM((K, H), jnp.float32),
            pltpu.VMEM_SHARED((NP, H), jnp.float32),
            pltpu.SemaphoreType.DMA,
            pltpu.SemaphoreType.DMA,
        ))


def _make_sc_degree(NP, CH, K):
    RPT = NP // _NS

    def body(dst_hbm, znd_hbm, ones_hbm, deg_out, dstv, onesv, degsp):
        c = lax.axis_index("c")
        s = lax.axis_index("s")
        wid = c * _NS + s
        pltpu.sync_copy(znd_hbm.at[pl.ds(s * RPT, RPT)],
                        degsp.at[pl.ds(s * RPT, RPT)])
        pltpu.sync_copy(dst_hbm.at[wid], dstv)
        pltpu.sync_copy(ones_hbm, onesv)
        plsc.subcore_barrier()

        @pl.loop(0, CH)
        def _chunk(j):
            pltpu.sync_copy(onesv, degsp.at[dstv.at[j]], add=True)

        plsc.subcore_barrier()
        pltpu.sync_copy(degsp.at[pl.ds(s * RPT, RPT)],
                        deg_out.at[c, pl.ds(s * RPT, RPT)])

    return pl.kernel(
        body,
        out_type=jax.ShapeDtypeStruct((_NC, NP, _DW), jnp.float32),
        mesh=_sc_mesh(),
        scratch_types=(
            pltpu.VMEM((CH, K), jnp.int32),
            pltpu.VMEM((K, _DW), jnp.float32),
            pltpu.VMEM_SHARED((NP, _DW), jnp.float32),
        ))


def _enc_body(x_ref, w_ref, b_ref, o_ref):
    o_ref[...] = jnp.maximum(
        jnp.dot(x_ref[...], w_ref[...], preferred_element_type=jnp.float32)
        + b_ref[...], 0.0)


def _tc_encode(x, w, b, R):
    N, D = x.shape
    H = w.shape[1]
    return pl.pallas_call(
        _enc_body,
        grid=(N // R,),
        in_specs=[
            pl.BlockSpec((R, D), lambda i: (i, 0)),
            pl.BlockSpec((D, H), lambda i: (0, 0)),
            pl.BlockSpec((1, H), lambda i: (0, 0)),
        ],
        out_specs=pl.BlockSpec((R, H), lambda i: (i, 0)),
        out_shape=jax.ShapeDtypeStruct((N, H), jnp.float32),
    )(x, w, b.reshape(1, H))


def _gnn_update(h_ref, a0_ref, a1_ref, d0_ref, d1_ref, ws_ref, wn_ref, b_ref):
    deg = jnp.maximum(d0_ref[...][:, :1] + d1_ref[...][:, :1], 1.0)
    agg = (a0_ref[...] + a1_ref[...]) / deg
    return jnp.maximum(
        jnp.dot(h_ref[...], ws_ref[...], preferred_element_type=jnp.float32)
        + jnp.dot(agg, wn_ref[...], preferred_element_type=jnp.float32)
        + b_ref[...], 0.0)


def _layer_body(h_ref, a0_ref, a1_ref, d0_ref, d1_ref, ws_ref, wn_ref, b_ref,
                o_ref):
    o_ref[...] = _gnn_update(h_ref, a0_ref, a1_ref, d0_ref, d1_ref,
                             ws_ref, wn_ref, b_ref)


def _layer_specs(R, H):
    return [
        pl.BlockSpec((R, H), lambda i: (i, 0)),
        pl.BlockSpec((R, H), lambda i: (i, 0)),
        pl.BlockSpec((R, H), lambda i: (i, 0)),
        pl.BlockSpec((R, _DW), lambda i: (i, 0)),
        pl.BlockSpec((R, _DW), lambda i: (i, 0)),
        pl.BlockSpec((H, H), lambda i: (0, 0)),
        pl.BlockSpec((H, H), lambda i: (0, 0)),
        pl.BlockSpec((1, H), lambda i: (0, 0)),
    ]


def _tc_layer(h, aggp, degp, ws, wn, b, R):
    N, H = h.shape
    return pl.pallas_call(
        _layer_body,
        grid=(N // R,),
        in_specs=_layer_specs(R, H),
        out_specs=pl.BlockSpec((R, H), lambda i: (i, 0)),
        out_shape=jax.ShapeDtypeStruct((N, H), jnp.float32),
    )(h, aggp[0], aggp[1], degp[0], degp[1], ws, wn, b.reshape(1, H))


def _final_body(h_ref, a0_ref, a1_ref, d0_ref, d1_ref, ws_ref, wn_ref, b_ref,
                ad_ref, wad_ref, bad_ref, mk_ref, o_ref):
    h3 = _gnn_update(h_ref, a0_ref, a1_ref, d0_ref, d1_ref,
                     ws_ref, wn_ref, b_ref)
    H = h3.shape[1]
    ad = jnp.maximum(
        jnp.dot(ad_ref[...], wad_ref[...], preferred_element_type=jnp.float32)
        + bad_ref[...], 0.0)
    logits = lax.dot_general(ad, h3, (((1,), (1,)), ((), ())),
                             preferred_element_type=jnp.float32)
    logits = logits * (1.0 / math.sqrt(H))
    o_ref[...] = jnp.where(mk_ref[...][0] > 0.0, logits,
                           jnp.float32(-1e9))[None]


def _tc_final(h, aggp, degp, ws, wn, b, ad, wad, bad, mask_f, R):
    N, H = h.shape
    A = ad.shape[1]
    G = N // R
    specs = _layer_specs(R, H) + [
        pl.BlockSpec((1, A), lambda i: (0, 0)),
        pl.BlockSpec((A, H), lambda i: (0, 0)),
        pl.BlockSpec((1, H), lambda i: (0, 0)),
        pl.BlockSpec((1, 1, R), lambda i: (i, 0, 0)),
    ]
    out = pl.pallas_call(
        _final_body,
        grid=(G,),
        in_specs=specs,
        out_specs=pl.BlockSpec((1, 1, R), lambda i: (i, 0, 0)),
        out_shape=jax.ShapeDtypeStruct((G, 1, R), jnp.float32),
    )(h, aggp[0], aggp[1], degp[0], degp[1], ws, wn, b.reshape(1, H),
      ad, wad, bad.reshape(1, H), mask_f.reshape(G, 1, R))
    return out.reshape(N)


def kernel(graph_nodes, graph_edge_links, mask, current_ad, W_enc, b_enc,
           W_self1, W_nbr1, b1, W_self2, W_nbr2, b2, W_ad, b_ad):
    B, N, D = graph_nodes.shape
    H = W_enc.shape[1]
    E = graph_edge_links.shape[2]
    NW = _NC * _NS
    assert E % NW == 0 and N % _NS == 0
    EW = E // NW
    K = _pick_chunk(EW)
    CH = EW // K
    R = 2000 if N % 2000 == 0 else 8
    NP = ((N + 127) // 128) * 128

    znh = jnp.zeros((NP, H), jnp.float32)
    znd = jnp.zeros((NP, _DW), jnp.float32)
    ones = jnp.ones((K, _DW), jnp.float32)

    sc_agg = _make_sc_aggregate(NP, H, CH, K)
    sc_deg = _make_sc_degree(NP, CH, K)

    outs = []
    for bi in range(B):
        x = graph_nodes[bi]
        src = graph_edge_links[bi, 0]
        dst = graph_edge_links[bi, 1]
        dst2 = dst.reshape(NW, CH, K)
        pk2 = (src | (dst << 16)).reshape(NW, CH, K)
        mask_f = mask[bi].astype(jnp.float32)

        h1 = _tc_encode(x, W_enc, b_enc, R)
        degp = sc_deg(dst2, znd, ones)
        aggp1 = sc_agg(h1, pk2, znh)
        h2 = _tc_layer(h1, aggp1, degp, W_self1, W_nbr1, b1, R)
        aggp2 = sc_agg(h2, pk2, znh)
        logits = _tc_final(h2, aggp2, degp, W_self2, W_nbr2, b2,
                           current_ad[bi:bi + 1], W_ad, b_ad, mask_f, R)
        outs.append(logits)
    return jnp.stack(outs, axis=0)


# speedup vs baseline: 11.0878x; 1.1513x over previous
import functools
import math

import jax
import jax.numpy as jnp
from jax import lax
from jax.experimental import pallas as pl
from jax.experimental.pallas import tpu as pltpu
import jax.experimental.pallas.tpu_sc as plsc

_NC = 2
_NS = 16
_DW = 128


def _pick_chunk(edges_per_worker):
    k = 0
    for cand in range(8, 129, 8):
        if edges_per_worker % cand == 0:
            k = cand
    return k


def _sc_mesh():
    return plsc.VectorSubcoreMesh(core_axis_name="c", subcore_axis_name="s",
                                  num_cores=_NC, num_subcores=_NS)


def _make_sc_aggregate(NP, H, CH, K):
    RPT = NP // _NS

    def body(h_hbm, pk_hbm, znh_hbm, agg_out,
             pkc0, pkc1, pkc2, srcc0, srcc1, srcc2, dstc0, dstc1, dstc2,
             rows0, rows1, rows2, aggsp, sem_i, sem_g, sem_s):
        c = lax.axis_index("c")
        s = lax.axis_index("s")
        wid = c * _NS + s
        base = wid * CH
        pkc = (pkc0, pkc1, pkc2)
        srcc = (srcc0, srcc1, srcc2)
        dstc = (dstc0, dstc1, dstc2)
        rows = (rows0, rows1, rows2)

        pltpu.sync_copy(znh_hbm.at[pl.ds(s * RPT, RPT)],
                        aggsp.at[pl.ds(s * RPT, RPT)])
        plsc.subcore_barrier()

        def fire_idx(t, b):
            pltpu.async_copy(pk_hbm.at[base + t], pkc[b], sem_i)

        def wait_idx():
            pltpu.make_async_copy(pk_hbm.at[base], pkc[0], sem_i).wait()

        def unpack(b):
            for t in range(K // 16):
                v = pkc[b][pl.ds(t * 16, 16)]
                srcc[b][pl.ds(t * 16, 16)] = lax.bitwise_and(v, 0xFFFF)
                dstc[b][pl.ds(t * 16, 16)] = lax.shift_right_logical(v, 16)

        def fire_gather(b):
            pltpu.async_copy(h_hbm.at[srcc[b]], rows[b], sem_g)

        def wait_gather():
            pltpu.make_async_copy(h_hbm.at[srcc[0]], rows[0], sem_g).wait()

        def fire_scatter(b):
            pltpu.async_copy(rows[b], aggsp.at[dstc[b]], sem_s, add=True)

        def wait_scatter():
            pltpu.make_async_copy(rows[0], aggsp.at[dstc[0]], sem_s).wait()

        for t in range(3):
            fire_idx(t, t)
        wait_idx()
        unpack(0)
        wait_idx()
        unpack(1)
        fire_gather(0)
        fire_gather(1)

        @pl.loop(0, CH, step=3)
        def _chunk(j):
            for b in range(3):
                jj = j + b

                @pl.when(jj < CH)
                def _():
                    wait_gather()
                    fire_scatter(b)

                    @pl.when(jj >= 1)
                    def _():
                        wait_scatter()

                    @pl.when(jj + 2 < CH)
                    def _():
                        wait_idx()
                        unpack((b + 2) % 3)
                        fire_gather((b + 2) % 3)

                    @pl.when(jj + 3 < CH)
                    def _():
                        fire_idx(jj + 3, b)

        wait_scatter()

        plsc.subcore_barrier()
        pltpu.sync_copy(aggsp.at[pl.ds(s * RPT, RPT)],
                        agg_out.at[c, pl.ds(s * RPT, RPT)])

    return pl.kernel(
        body,
        out_type=jax.ShapeDtypeStruct((_NC, NP, H), jnp.float32),
        mesh=_sc_mesh(),
        scratch_types=(
            pltpu.VMEM((K,), jnp.int32),
            pltpu.VMEM((K,), jnp.int32),
            pltpu.VMEM((K,), jnp.int32),
            pltpu.VMEM((K,), jnp.int32),
            pltpu.VMEM((K,), jnp.int32),
            pltpu.VMEM((K,), jnp.int32),
            pltpu.VMEM((K,), jnp.int32),
            pltpu.VMEM((K,), jnp.int32),
            pltpu.VMEM((K,), jnp.int32),
            pltpu.VMEM((K, H), jnp.float32),
            pltpu.VMEM((K, H), jnp.float32),
            pltpu.VMEM((K, H), jnp.float32),
            pltpu.VMEM_SHARED((NP, H), jnp.float32),
            pltpu.SemaphoreType.DMA,
            pltpu.SemaphoreType.DMA,
            pltpu.SemaphoreType.DMA,
        ))


def _make_sc_degree(NP, CH, K):
    RPT = NP // _NS

    W = 4

    def body(dst_hbm, znd_hbm, ones_hbm, deg_out, dstv, onesv, degsp, sem_s):
        c = lax.axis_index("c")
        s = lax.axis_index("s")
        wid = c * _NS + s
        pltpu.sync_copy(znd_hbm.at[pl.ds(s * RPT, RPT)],
                        degsp.at[pl.ds(s * RPT, RPT)])
        pltpu.sync_copy(dst_hbm.at[wid], dstv)
        pltpu.sync_copy(ones_hbm, onesv)
        plsc.subcore_barrier()

        @pl.loop(0, CH)
        def _chunk(j):
            pltpu.async_copy(onesv, degsp.at[dstv.at[j]], sem_s, add=True)

            @pl.when(j >= W)
            def _():
                pltpu.make_async_copy(onesv, degsp.at[dstv.at[0]],
                                      sem_s).wait()

        @pl.loop(0, min(W, CH))
        def _drain(j):
            pltpu.make_async_copy(onesv, degsp.at[dstv.at[0]], sem_s).wait()

        plsc.subcore_barrier()
        pltpu.sync_copy(degsp.at[pl.ds(s * RPT, RPT)],
                        deg_out.at[c, pl.ds(s * RPT, RPT)])

    return pl.kernel(
        body,
        out_type=jax.ShapeDtypeStruct((_NC, NP, _DW), jnp.float32),
        mesh=_sc_mesh(),
        scratch_types=(
            pltpu.VMEM((CH, K), jnp.int32),
            pltpu.VMEM((K, _DW), jnp.float32),
            pltpu.VMEM_SHARED((NP, _DW), jnp.float32),
            pltpu.SemaphoreType.DMA,
        ))


def _enc_body(x_ref, w_ref, b_ref, o_ref):
    o_ref[...] = jnp.maximum(
        jnp.dot(x_ref[...], w_ref[...], preferred_element_type=jnp.float32)
        + b_ref[...], 0.0)


def _tc_encode(x, w, b, R):
    N, D = x.shape
    H = w.shape[1]
    return pl.pallas_call(
        _enc_body,
        grid=(N // R,),
        in_specs=[
            pl.BlockSpec((R, D), lambda i: (i, 0)),
            pl.BlockSpec((D, H), lambda i: (0, 0)),
            pl.BlockSpec((1, H), lambda i: (0, 0)),
        ],
        out_specs=pl.BlockSpec((R, H), lambda i: (i, 0)),
        out_shape=jax.ShapeDtypeStruct((N, H), jnp.float32),
    )(x, w, b.reshape(1, H))


def _gnn_update(h_ref, a0_ref, a1_ref, d0_ref, d1_ref, ws_ref, wn_ref, b_ref):
    deg = jnp.maximum(d0_ref[...][:, :1] + d1_ref[...][:, :1], 1.0)
    agg = (a0_ref[...] + a1_ref[...]) / deg
    return jnp.maximum(
        jnp.dot(h_ref[...], ws_ref[...], preferred_element_type=jnp.float32)
        + jnp.dot(agg, wn_ref[...], preferred_element_type=jnp.float32)
        + b_ref[...], 0.0)


def _layer_body(h_ref, a0_ref, a1_ref, d0_ref, d1_ref, ws_ref, wn_ref, b_ref,
                o_ref):
    o_ref[...] = _gnn_update(h_ref, a0_ref, a1_ref, d0_ref, d1_ref,
                             ws_ref, wn_ref, b_ref)


def _layer_specs(R, H):
    return [
        pl.BlockSpec((R, H), lambda i: (i, 0)),
        pl.BlockSpec((R, H), lambda i: (i, 0)),
        pl.BlockSpec((R, H), lambda i: (i, 0)),
        pl.BlockSpec((R, _DW), lambda i: (i, 0)),
        pl.BlockSpec((R, _DW), lambda i: (i, 0)),
        pl.BlockSpec((H, H), lambda i: (0, 0)),
        pl.BlockSpec((H, H), lambda i: (0, 0)),
        pl.BlockSpec((1, H), lambda i: (0, 0)),
    ]


def _tc_layer(h, aggp, degp, ws, wn, b, R):
    N, H = h.shape
    return pl.pallas_call(
        _layer_body,
        grid=(N // R,),
        in_specs=_layer_specs(R, H),
        out_specs=pl.BlockSpec((R, H), lambda i: (i, 0)),
        out_shape=jax.ShapeDtypeStruct((N, H), jnp.float32),
    )(h, aggp[0], aggp[1], degp[0], degp[1], ws, wn, b.reshape(1, H))


def _final_body(h_ref, a0_ref, a1_ref, d0_ref, d1_ref, ws_ref, wn_ref, b_ref,
                ad_ref, wad_ref, bad_ref, mk_ref, o_ref):
    h3 = _gnn_update(h_ref, a0_ref, a1_ref, d0_ref, d1_ref,
                     ws_ref, wn_ref, b_ref)
    H = h3.shape[1]
    ad = jnp.maximum(
        jnp.dot(ad_ref[...], wad_ref[...], preferred_element_type=jnp.float32)
        + bad_ref[...], 0.0)
    logits = lax.dot_general(ad, h3, (((1,), (1,)), ((), ())),
                             preferred_element_type=jnp.float32)
    logits = logits * (1.0 / math.sqrt(H))
    o_ref[...] = jnp.where(mk_ref[...][0] > 0.0, logits,
                           jnp.float32(-1e9))[None]


def _tc_final(h, aggp, degp, ws, wn, b, ad, wad, bad, mask_f, R):
    N, H = h.shape
    A = ad.shape[1]
    G = N // R
    specs = _layer_specs(R, H) + [
        pl.BlockSpec((1, A), lambda i: (0, 0)),
        pl.BlockSpec((A, H), lambda i: (0, 0)),
        pl.BlockSpec((1, H), lambda i: (0, 0)),
        pl.BlockSpec((1, 1, R), lambda i: (i, 0, 0)),
    ]
    out = pl.pallas_call(
        _final_body,
        grid=(G,),
        in_specs=specs,
        out_specs=pl.BlockSpec((1, 1, R), lambda i: (i, 0, 0)),
        out_shape=jax.ShapeDtypeStruct((G, 1, R), jnp.float32),
    )(h, aggp[0], aggp[1], degp[0], degp[1], ws, wn, b.reshape(1, H),
      ad, wad, bad.reshape(1, H), mask_f.reshape(G, 1, R))
    return out.reshape(N)


def kernel(graph_nodes, graph_edge_links, mask, current_ad, W_enc, b_enc,
           W_self1, W_nbr1, b1, W_self2, W_nbr2, b2, W_ad, b_ad):
    B, N, D = graph_nodes.shape
    H = W_enc.shape[1]
    E = graph_edge_links.shape[2]
    NW = _NC * _NS
    assert E % NW == 0 and N % _NS == 0
    EW = E // NW
    K = _pick_chunk(EW)
    CH = EW // K
    R = 2000 if N % 2000 == 0 else 8
    NP = ((N + 127) // 128) * 128

    znh = jnp.zeros((NP, H), jnp.float32)
    znd = jnp.zeros((NP, _DW), jnp.float32)
    ones = jnp.ones((K, _DW), jnp.float32)

    sc_agg = _make_sc_aggregate(NP, H, CH, K)
    sc_deg = _make_sc_degree(NP, CH, K)

    outs = []
    for bi in range(B):
        x = graph_nodes[bi]
        src = graph_edge_links[bi, 0]
        dst = graph_edge_links[bi, 1]
        dst2 = dst.reshape(NW, CH, K)
        pk2 = (src | (dst << 16)).reshape(E // K, K)
        mask_f = mask[bi].astype(jnp.float32)

        h1 = _tc_encode(x, W_enc, b_enc, R)
        degp = sc_deg(dst2, znd, ones)
        aggp1 = sc_agg(h1, pk2, znh)
        h2 = _tc_layer(h1, aggp1, degp, W_self1, W_nbr1, b1, R)
        aggp2 = sc_agg(h2, pk2, znh)
        logits = _tc_final(h2, aggp2, degp, W_self2, W_nbr2, b2,
                           current_ad[bi:bi + 1], W_ad, b_ad, mask_f, R)
        outs.append(logits)
    return jnp.stack(outs, axis=0)
